# baseline (speedup 1.0000x reference)
import functools
import numpy as np
import jax
import jax.numpy as jnp
from jax.experimental import pallas as pl
from jax.experimental.pallas import tpu as pltpu

N = 2048
HID = 16
C = 8
GROUPS = 4
RADIUS = 9.0
TB = 128
NB = N // TB


def _lrelu(x):
    return jnp.maximum(x, 0.2 * x)


def _group_norm(v, gamma, beta):
    cs = jnp.sum(v, axis=0, keepdims=True)
    css = jnp.sum(v * v, axis=0, keepdims=True)
    cnt = float((HID // GROUPS) * N)
    mean_parts = []
    inv_parts = []
    gsz = HID // GROUPS
    for g in range(GROUPS):
        m = jnp.sum(cs[:, g * gsz:(g + 1) * gsz]) / cnt
        ex2 = jnp.sum(css[:, g * gsz:(g + 1) * gsz]) / cnt
        var = ex2 - m * m
        inv = jax.lax.rsqrt(var + 1e-5)
        ones = jnp.ones((1, gsz), jnp.float32)
        mean_parts.append(m * ones)
        inv_parts.append(inv * ones)
    mean_ch = jnp.concatenate(mean_parts, axis=1)
    inv_ch = jnp.concatenate(inv_parts, axis=1)
    return (v - mean_ch) * (inv_ch * gamma) + beta


def _layer_kernel(
    x_ref, pts_blk, nuv_blk, ptsT_ref, sqK_ref, nrmK_ref,
    in_w1, in_b1, in_w2, in_b2, gni_w, gni_b,
    cw1_ref, cb1row_ref, cw2T_ref, cb2_ref,
    out_w1, out_b1, out_w2, out_b2, gno_w, gno_b,
    lin_w1, lin_b1, lin_w2, lin_b2, lt_w, lt_b,
    out_ref,
    f_scr, conv_scr,
):
    i = pl.program_id(0)

    @pl.when(i == 0)
    def _pre():
        h = x_ref[...] @ in_w1[...] + in_b1[...]
        h = _lrelu(h)
        h = _lrelu(h @ in_w2[...] + in_b2[...])
        f_scr[...] = _group_norm(h, gni_w[...], gni_b[...])

    pi = pts_blk[...]
    nuvi = nuv_blk[...]
    ni = nuvi[:, 0:3]

    pi_hi = jax.lax.convert_element_type(
        jax.lax.convert_element_type(pi, jnp.bfloat16), jnp.float32)
    pi_lo = pi - pi_hi
    sq_pi = jnp.sum(pi * pi, axis=1, keepdims=True)
    sqpi_hi = jax.lax.convert_element_type(
        jax.lax.convert_element_type(sq_pi, jnp.bfloat16), jnp.float32)
    sqpi_lo = sq_pi - sqpi_hi
    ones2 = jnp.ones((TB, 1), jnp.float32)
    lhs_sq = jnp.concatenate(
        [-2.0 * pi_hi, -2.0 * pi_lo, -2.0 * pi_hi,
         sqpi_hi, sqpi_lo, ones2, ones2], axis=1)
    sqdist = jnp.dot(lhs_sq.astype(jnp.bfloat16), sqK_ref[...],
                     preferred_element_type=jnp.float32)

    ni_hi = jax.lax.convert_element_type(
        jax.lax.convert_element_type(ni, jnp.bfloat16), jnp.float32)
    ni_lo = ni - ni_hi
    lhs_dot = jnp.concatenate([ni_hi, ni_lo, ni_hi], axis=1)
    dot = jnp.dot(lhs_dot.astype(jnp.bfloat16), nrmK_ref[...],
                  preferred_element_type=jnp.float32)

    scal = 2.0 - dot
    rho2 = sqdist * (scal * scal)
    window = jnp.exp(-rho2)

    vparts = []
    for c in range(C):
        vc = (cw1_ref[c:c + 1, 0:1] * nuvi[:, 0:3]
              + cw1_ref[c:c + 1, 1:2] * nuvi[:, 3:6]
              + cw1_ref[c:c + 1, 2:3] * nuvi[:, 6:9])
        vparts.append(vc)
    V = jnp.concatenate(vparts, axis=0)
    pi_rep = jnp.concatenate([pi] * C, axis=0)
    bias = cb1row_ref[...] - jnp.sum(V * pi_rep, axis=1, keepdims=True)

    Vb = jnp.concatenate([V, bias], axis=1)
    W = jnp.dot(Vb.astype(jnp.bfloat16), ptsT_ref[...],
                preferred_element_type=jnp.float32).astype(jnp.bfloat16)
    win_bf = window.astype(jnp.bfloat16)
    Yr = jnp.maximum(W, jnp.bfloat16(0.0))
    Z = (Yr.reshape(C, TB, N) * win_bf[None, :, :]).reshape(C * TB, N)

    f_bf = f_scr[...].astype(jnp.bfloat16)
    S = jnp.dot(Z, f_bf, preferred_element_type=jnp.float32)
    T = jnp.dot(win_bf, f_bf, preferred_element_type=jnp.float32)
    acc = T * cb2_ref[...]
    for c in range(C):
        acc = acc + S[c * TB:(c + 1) * TB, :] * cw2T_ref[c:c + 1, :]
    conv_scr[pl.ds(i * TB, TB), :] = acc

    @pl.when(i == NB - 1)
    def _post():
        o = _lrelu(conv_scr[...] @ out_w1[...] + out_b1[...])
        o = _lrelu(o @ out_w2[...] + out_b2[...])
        y = _group_norm(o, gno_w[...], gno_b[...])
        xi = jnp.maximum(y @ lin_w1[...] + lin_b1[...], 0.0)
        xi = xi @ lin_w2[...] + lin_b2[...]
        out_ref[...] = x_ref[...] @ lt_w[...] + lt_b[...] + xi


def _full(shape):
    nd = len(shape)
    return pl.BlockSpec(shape, lambda i, _nd=nd: (0,) * _nd)


def _layer(x, pts_s, ptsT_s, sqK, nrmK, nuv9, p):
    r2 = lambda a: a.reshape(1, -1)
    cb1row = jnp.repeat(p['cb1'], TB).reshape(C * TB, 1)
    inputs = [
        x, pts_s, nuv9, ptsT_s, sqK, nrmK,
        p['in_w1'].T, r2(p['in_b1']), p['in_w2'].T, r2(p['in_b2']),
        r2(p['gn_in_w']), r2(p['gn_in_b']),
        p['cw1'], cb1row, p['cw2'].T, r2(p['cb2']),
        p['out_w1'].T, r2(p['out_b1']), p['out_w2'].T, r2(p['out_b2']),
        r2(p['gn_out_w']), r2(p['gn_out_b']),
        p['lin_w1'].T, r2(p['lin_b1']), p['lin_w2'].T, r2(p['lin_b2']),
        p['lt_w'].T, r2(p['lt_b']),
    ]
    in_specs = [_full(a.shape) for a in inputs]
    in_specs[1] = pl.BlockSpec((TB, 3), lambda i: (i, 0))
    in_specs[2] = pl.BlockSpec((TB, 9), lambda i: (i, 0))
    return pl.pallas_call(
        _layer_kernel,
        grid=(NB,),
        in_specs=in_specs,
        out_specs=_full((N, HID)),
        out_shape=jax.ShapeDtypeStruct((N, HID), jnp.float32),
        scratch_shapes=[
            pltpu.VMEM((N, HID), jnp.float32),
            pltpu.VMEM((N, HID), jnp.float32),
        ],
        compiler_params=pltpu.CompilerParams(
            dimension_semantics=("arbitrary",),
        ),
    )(*inputs)


def _hi_lo(a):
    hi = a.astype(jnp.bfloat16).astype(jnp.float32)
    return hi, a - hi


@jax.jit
def kernel(features, points, nuv, params):
    scale = 1.0 / np.sqrt(2.0) / RADIUS
    pts_s = points * scale
    ptsT_s = pts_s.T
    nrmT = nuv[:, 0, :].T
    nuv9 = nuv.reshape(N, 9)
    q_hi, q_lo = _hi_lo(ptsT_s)
    sq_pts = jnp.sum(ptsT_s * ptsT_s, axis=0, keepdims=True)
    sq_hi, sq_lo = _hi_lo(sq_pts)
    ones_row = jnp.ones((1, N), jnp.float32)
    sqK = jnp.concatenate(
        [q_hi, q_hi, q_lo, ones_row, ones_row, sq_hi, sq_lo],
        axis=0).astype(jnp.bfloat16)
    n_hi, n_lo = _hi_lo(nrmT)
    nrmK = jnp.concatenate([n_hi, n_hi, n_lo],
                           axis=0).astype(jnp.bfloat16)
    ptsT_bf = jnp.concatenate([ptsT_s, jnp.ones((1, N), jnp.float32)],
                              axis=0).astype(jnp.bfloat16)
    x = features
    for p in params:
        x = _layer(x, pts_s, ptsT_bf, sqK, nrmK, nuv9, p)
    return x


# speedup vs baseline: 1.1005x; 1.1005x over previous
import functools
import numpy as np
import jax
import jax.numpy as jnp
from jax.experimental import pallas as pl
from jax.experimental.pallas import tpu as pltpu

N = 2048
HID = 16
C = 8
GROUPS = 4
RADIUS = 9.0
TB = 128
NB = N // TB


def _lrelu(x):
    return jnp.maximum(x, 0.2 * x)


def _group_norm(v, gamma, beta):
    cs = jnp.sum(v, axis=0, keepdims=True)
    css = jnp.sum(v * v, axis=0, keepdims=True)
    cnt = float((HID // GROUPS) * N)
    mean_parts = []
    inv_parts = []
    gsz = HID // GROUPS
    for g in range(GROUPS):
        m = jnp.sum(cs[:, g * gsz:(g + 1) * gsz]) / cnt
        ex2 = jnp.sum(css[:, g * gsz:(g + 1) * gsz]) / cnt
        var = ex2 - m * m
        inv = jax.lax.rsqrt(var + 1e-5)
        ones = jnp.ones((1, gsz), jnp.float32)
        mean_parts.append(m * ones)
        inv_parts.append(inv * ones)
    mean_ch = jnp.concatenate(mean_parts, axis=1)
    inv_ch = jnp.concatenate(inv_parts, axis=1)
    return (v - mean_ch) * (inv_ch * gamma) + beta


def _layer_kernel(
    x_ref, pts_blk, nuv_blk, ptsT_ref, sqrow_ref, nrmT_ref,
    in_w1, in_b1, in_w2, in_b2, gni_w, gni_b,
    cw1_ref, cb1row_ref, cw2T_ref, cb2_ref,
    out_w1, out_b1, out_w2, out_b2, gno_w, gno_b,
    lin_w1, lin_b1, lin_w2, lin_b2, lt_w, lt_b,
    out_ref,
    f_scr, conv_scr,
):
    i = pl.program_id(0)

    @pl.when(i == 0)
    def _pre():
        h = x_ref[...] @ in_w1[...] + in_b1[...]
        h = _lrelu(h)
        h = _lrelu(h @ in_w2[...] + in_b2[...])
        f_scr[...] = _group_norm(h, gni_w[...], gni_b[...])

    pi = pts_blk[...]
    nuvi = nuv_blk[...]
    ni = nuvi[:, 0:3]

    pdot = jnp.dot(pi, ptsT_ref[...],
                   precision=jax.lax.Precision.HIGHEST)
    sq_pi = jnp.sum(pi * pi, axis=1, keepdims=True)
    sqdist = sq_pi + sqrow_ref[...] - 2.0 * pdot

    dot = jnp.dot(ni, nrmT_ref[...],
                  precision=jax.lax.Precision.HIGHEST)

    scal = 2.0 - dot
    rho2 = sqdist * (scal * scal)
    window = jnp.exp(-rho2)

    vparts = []
    for c in range(C):
        vc = (cw1_ref[c:c + 1, 0:1] * nuvi[:, 0:3]
              + cw1_ref[c:c + 1, 1:2] * nuvi[:, 3:6]
              + cw1_ref[c:c + 1, 2:3] * nuvi[:, 6:9])
        vparts.append(vc)
    V = jnp.concatenate(vparts, axis=0)
    pi_rep = jnp.concatenate([pi] * C, axis=0)
    bias = cb1row_ref[...] - jnp.sum(V * pi_rep, axis=1, keepdims=True)

    W = jnp.dot(V, ptsT_ref[...])
    Yr = jnp.maximum(W + bias, 0.0)
    Z = (Yr.reshape(C, TB, N) * window[None, :, :]).reshape(C * TB, N)
    ZW = jnp.concatenate([Z, window], axis=0)

    S = jnp.dot(ZW, f_scr[...])
    acc = S[C * TB:, :] * cb2_ref[...]
    for c in range(C):
        acc = acc + S[c * TB:(c + 1) * TB, :] * cw2T_ref[c:c + 1, :]
    conv_scr[pl.ds(i * TB, TB), :] = acc

    @pl.when(i == NB - 1)
    def _post():
        o = _lrelu(conv_scr[...] @ out_w1[...] + out_b1[...])
        o = _lrelu(o @ out_w2[...] + out_b2[...])
        y = _group_norm(o, gno_w[...], gno_b[...])
        xi = jnp.maximum(y @ lin_w1[...] + lin_b1[...], 0.0)
        xi = xi @ lin_w2[...] + lin_b2[...]
        out_ref[...] = x_ref[...] @ lt_w[...] + lt_b[...] + xi


def _full(shape):
    nd = len(shape)
    return pl.BlockSpec(shape, lambda i, _nd=nd: (0,) * _nd)


def _layer(x, pts_s, ptsT_s, sqrow, nrmT, nuv9, p):
    r2 = lambda a: a.reshape(1, -1)
    cb1row = jnp.repeat(p['cb1'], TB).reshape(C * TB, 1)
    inputs = [
        x, pts_s, nuv9, ptsT_s, sqrow, nrmT,
        p['in_w1'].T, r2(p['in_b1']), p['in_w2'].T, r2(p['in_b2']),
        r2(p['gn_in_w']), r2(p['gn_in_b']),
        p['cw1'], cb1row, p['cw2'].T, r2(p['cb2']),
        p['out_w1'].T, r2(p['out_b1']), p['out_w2'].T, r2(p['out_b2']),
        r2(p['gn_out_w']), r2(p['gn_out_b']),
        p['lin_w1'].T, r2(p['lin_b1']), p['lin_w2'].T, r2(p['lin_b2']),
        p['lt_w'].T, r2(p['lt_b']),
    ]
    in_specs = [_full(a.shape) for a in inputs]
    in_specs[1] = pl.BlockSpec((TB, 3), lambda i: (i, 0))
    in_specs[2] = pl.BlockSpec((TB, 9), lambda i: (i, 0))
    return pl.pallas_call(
        _layer_kernel,
        grid=(NB,),
        in_specs=in_specs,
        out_specs=_full((N, HID)),
        out_shape=jax.ShapeDtypeStruct((N, HID), jnp.float32),
        scratch_shapes=[
            pltpu.VMEM((N, HID), jnp.float32),
            pltpu.VMEM((N, HID), jnp.float32),
        ],
        compiler_params=pltpu.CompilerParams(
            dimension_semantics=("arbitrary",),
        ),
    )(*inputs)


@jax.jit
def kernel(features, points, nuv, params):
    scale = 1.0 / np.sqrt(2.0) / RADIUS
    pts_s = points * scale
    ptsT_s = pts_s.T
    nrmT = nuv[:, 0, :].T
    nuv9 = nuv.reshape(N, 9)
    sqrow = jnp.sum(ptsT_s * ptsT_s, axis=0, keepdims=True)
    x = features
    for p in params:
        x = _layer(x, pts_s, ptsT_s, sqrow, nrmT, nuv9, p)
    return x


# speedup vs baseline: 1.3782x; 1.2524x over previous
import functools
import numpy as np
import jax
import jax.numpy as jnp
from jax.experimental import pallas as pl
from jax.experimental.pallas import tpu as pltpu

N = 2048
HID = 16
C = 8
GROUPS = 4
RADIUS = 9.0
TB = 128
NB = N // TB


def _lrelu(x):
    return jnp.maximum(x, 0.2 * x)


def _group_norm(v, gamma, beta):
    cs = jnp.sum(v, axis=0, keepdims=True)
    css = jnp.sum(v * v, axis=0, keepdims=True)
    cnt = float((HID // GROUPS) * N)
    mean_parts = []
    inv_parts = []
    gsz = HID // GROUPS
    for g in range(GROUPS):
        m = jnp.sum(cs[:, g * gsz:(g + 1) * gsz]) / cnt
        ex2 = jnp.sum(css[:, g * gsz:(g + 1) * gsz]) / cnt
        var = ex2 - m * m
        inv = jax.lax.rsqrt(var + 1e-5)
        ones = jnp.ones((1, gsz), jnp.float32)
        mean_parts.append(m * ones)
        inv_parts.append(inv * ones)
    mean_ch = jnp.concatenate(mean_parts, axis=1)
    inv_ch = jnp.concatenate(inv_parts, axis=1)
    return (v - mean_ch) * (inv_ch * gamma) + beta


def _layer_kernel(
    x_ref, pts_blk, nuv_blk, ptsT_ref, sqrow_ref, nrmT_ref,
    in_w1, in_b1, in_w2, in_b2, gni_w, gni_b,
    cw1_ref, cb1row_ref, cw2T_ref, cb2_ref,
    out_w1, out_b1, out_w2, out_b2, gno_w, gno_b,
    lin_w1, lin_b1, lin_w2, lin_b2, lt_w, lt_b,
    out_ref,
    f_scr, conv_scr,
):
    i = pl.program_id(0)

    @pl.when(i == 0)
    def _pre():
        h = x_ref[...] @ in_w1[...] + in_b1[...]
        h = _lrelu(h)
        h = _lrelu(h @ in_w2[...] + in_b2[...])
        f_scr[...] = _group_norm(h, gni_w[...], gni_b[...])

    pi = pts_blk[...]
    nuvi = nuv_blk[...]
    ni = nuvi[:, 0:3]

    d0 = pi[:, 0:1] - ptsT_ref[0:1, :]
    d1 = pi[:, 1:2] - ptsT_ref[1:2, :]
    d2 = pi[:, 2:3] - ptsT_ref[2:3, :]
    sqdist = d0 * d0 + d1 * d1 + d2 * d2

    dot = (ni[:, 0:1] * nrmT_ref[0:1, :]
           + ni[:, 1:2] * nrmT_ref[1:2, :]
           + ni[:, 2:3] * nrmT_ref[2:3, :])

    scal = 2.0 - dot
    rho2 = sqdist * (scal * scal)
    window = jnp.exp(-rho2)

    vparts = []
    for c in range(C):
        vc = (cw1_ref[c:c + 1, 0:1] * nuvi[:, 0:3]
              + cw1_ref[c:c + 1, 1:2] * nuvi[:, 3:6]
              + cw1_ref[c:c + 1, 2:3] * nuvi[:, 6:9])
        vparts.append(vc)
    V = jnp.concatenate(vparts, axis=0)
    pi_rep = jnp.concatenate([pi] * C, axis=0)
    bias = cb1row_ref[...] - jnp.sum(V * pi_rep, axis=1, keepdims=True)

    W = jnp.dot(V, ptsT_ref[...])
    Yr = jnp.maximum(W + bias, 0.0)
    Z = (Yr.reshape(C, TB, N) * window[None, :, :]).reshape(C * TB, N)
    ZW = jnp.concatenate([Z, window], axis=0)

    S = jnp.dot(ZW, f_scr[...])
    acc = S[C * TB:, :] * cb2_ref[...]
    for c in range(C):
        acc = acc + S[c * TB:(c + 1) * TB, :] * cw2T_ref[c:c + 1, :]
    conv_scr[pl.ds(i * TB, TB), :] = acc

    @pl.when(i == NB - 1)
    def _post():
        o = _lrelu(conv_scr[...] @ out_w1[...] + out_b1[...])
        o = _lrelu(o @ out_w2[...] + out_b2[...])
        y = _group_norm(o, gno_w[...], gno_b[...])
        xi = jnp.maximum(y @ lin_w1[...] + lin_b1[...], 0.0)
        xi = xi @ lin_w2[...] + lin_b2[...]
        out_ref[...] = x_ref[...] @ lt_w[...] + lt_b[...] + xi


def _full(shape):
    nd = len(shape)
    return pl.BlockSpec(shape, lambda i, _nd=nd: (0,) * _nd)


def _layer(x, pts_s, ptsT_s, sqrow, nrmT, nuv9, p):
    r2 = lambda a: a.reshape(1, -1)
    cb1row = jnp.repeat(p['cb1'], TB).reshape(C * TB, 1)
    inputs = [
        x, pts_s, nuv9, ptsT_s, sqrow, nrmT,
        p['in_w1'].T, r2(p['in_b1']), p['in_w2'].T, r2(p['in_b2']),
        r2(p['gn_in_w']), r2(p['gn_in_b']),
        p['cw1'], cb1row, p['cw2'].T, r2(p['cb2']),
        p['out_w1'].T, r2(p['out_b1']), p['out_w2'].T, r2(p['out_b2']),
        r2(p['gn_out_w']), r2(p['gn_out_b']),
        p['lin_w1'].T, r2(p['lin_b1']), p['lin_w2'].T, r2(p['lin_b2']),
        p['lt_w'].T, r2(p['lt_b']),
    ]
    in_specs = [_full(a.shape) for a in inputs]
    in_specs[1] = pl.BlockSpec((TB, 3), lambda i: (i, 0))
    in_specs[2] = pl.BlockSpec((TB, 9), lambda i: (i, 0))
    return pl.pallas_call(
        _layer_kernel,
        grid=(NB,),
        in_specs=in_specs,
        out_specs=_full((N, HID)),
        out_shape=jax.ShapeDtypeStruct((N, HID), jnp.float32),
        scratch_shapes=[
            pltpu.VMEM((N, HID), jnp.float32),
            pltpu.VMEM((N, HID), jnp.float32),
        ],
        compiler_params=pltpu.CompilerParams(
            dimension_semantics=("arbitrary",),
        ),
    )(*inputs)


@jax.jit
def kernel(features, points, nuv, params):
    scale = 1.0 / np.sqrt(2.0) / RADIUS
    pts_s = points * scale
    ptsT_s = pts_s.T
    nrmT = nuv[:, 0, :].T
    nuv9 = nuv.reshape(N, 9)
    sqrow = jnp.sum(ptsT_s * ptsT_s, axis=0, keepdims=True)
    x = features
    for p in params:
        x = _layer(x, pts_s, ptsT_s, sqrow, nrmT, nuv9, p)
    return x


# speedup vs baseline: 1.4262x; 1.0348x over previous
import functools
import numpy as np
import jax
import jax.numpy as jnp
from jax.experimental import pallas as pl
from jax.experimental.pallas import tpu as pltpu

N = 2048
HID = 16
C = 8
GROUPS = 4
RADIUS = 9.0
TB = 256
NB = N // TB


def _lrelu(x):
    return jnp.maximum(x, 0.2 * x)


def _group_norm(v, gamma, beta):
    cs = jnp.sum(v, axis=0, keepdims=True)
    css = jnp.sum(v * v, axis=0, keepdims=True)
    cnt = float((HID // GROUPS) * N)
    mean_parts = []
    inv_parts = []
    gsz = HID // GROUPS
    for g in range(GROUPS):
        m = jnp.sum(cs[:, g * gsz:(g + 1) * gsz]) / cnt
        ex2 = jnp.sum(css[:, g * gsz:(g + 1) * gsz]) / cnt
        var = ex2 - m * m
        inv = jax.lax.rsqrt(var + 1e-5)
        ones = jnp.ones((1, gsz), jnp.float32)
        mean_parts.append(m * ones)
        inv_parts.append(inv * ones)
    mean_ch = jnp.concatenate(mean_parts, axis=1)
    inv_ch = jnp.concatenate(inv_parts, axis=1)
    return (v - mean_ch) * (inv_ch * gamma) + beta


def _layer_kernel(
    x_ref, pts_blk, nuv_blk, ptsT_ref, sqrow_ref, nrmT_ref,
    in_w1, in_b1, in_w2, in_b2, gni_w, gni_b,
    cw1_ref, cb1row_ref, cw2T_ref, cb2_ref,
    out_w1, out_b1, out_w2, out_b2, gno_w, gno_b,
    lin_w1, lin_b1, lin_w2, lin_b2, lt_w, lt_b,
    out_ref,
    f_scr, conv_scr,
):
    i = pl.program_id(0)

    @pl.when(i == 0)
    def _pre():
        h = x_ref[...] @ in_w1[...] + in_b1[...]
        h = _lrelu(h)
        h = _lrelu(h @ in_w2[...] + in_b2[...])
        f_scr[...] = _group_norm(h, gni_w[...], gni_b[...])

    pi = pts_blk[...]
    nuvi = nuv_blk[...]
    ni = nuvi[:, 0:3]

    d0 = pi[:, 0:1] - ptsT_ref[0:1, :]
    d1 = pi[:, 1:2] - ptsT_ref[1:2, :]
    d2 = pi[:, 2:3] - ptsT_ref[2:3, :]
    sqdist = d0 * d0 + d1 * d1 + d2 * d2

    dot = (ni[:, 0:1] * nrmT_ref[0:1, :]
           + ni[:, 1:2] * nrmT_ref[1:2, :]
           + ni[:, 2:3] * nrmT_ref[2:3, :])

    scal = 2.0 - dot
    rho2 = sqdist * (scal * scal)
    window = jnp.exp(-rho2)

    vparts = []
    for c in range(C):
        vc = (cw1_ref[c:c + 1, 0:1] * nuvi[:, 0:3]
              + cw1_ref[c:c + 1, 1:2] * nuvi[:, 3:6]
              + cw1_ref[c:c + 1, 2:3] * nuvi[:, 6:9])
        vparts.append(vc)
    V = jnp.concatenate(vparts, axis=0)
    pi_rep = jnp.concatenate([pi] * C, axis=0)
    bias = cb1row_ref[...] - jnp.sum(V * pi_rep, axis=1, keepdims=True)

    W = jnp.dot(V, ptsT_ref[...])
    Yr = jnp.maximum(W + bias, 0.0)
    Z = (Yr.reshape(C, TB, N) * window[None, :, :]).reshape(C * TB, N)
    ZW = jnp.concatenate([Z, window], axis=0)

    S = jnp.dot(ZW, f_scr[...])
    acc = S[C * TB:, :] * cb2_ref[...]
    for c in range(C):
        acc = acc + S[c * TB:(c + 1) * TB, :] * cw2T_ref[c:c + 1, :]
    conv_scr[pl.ds(i * TB, TB), :] = acc

    @pl.when(i == NB - 1)
    def _post():
        o = _lrelu(conv_scr[...] @ out_w1[...] + out_b1[...])
        o = _lrelu(o @ out_w2[...] + out_b2[...])
        y = _group_norm(o, gno_w[...], gno_b[...])
        xi = jnp.maximum(y @ lin_w1[...] + lin_b1[...], 0.0)
        xi = xi @ lin_w2[...] + lin_b2[...]
        out_ref[...] = x_ref[...] @ lt_w[...] + lt_b[...] + xi


def _full(shape):
    nd = len(shape)
    return pl.BlockSpec(shape, lambda i, _nd=nd: (0,) * _nd)


def _layer(x, pts_s, ptsT_s, sqrow, nrmT, nuv9, p):
    r2 = lambda a: a.reshape(1, -1)
    cb1row = jnp.repeat(p['cb1'], TB).reshape(C * TB, 1)
    inputs = [
        x, pts_s, nuv9, ptsT_s, sqrow, nrmT,
        p['in_w1'].T, r2(p['in_b1']), p['in_w2'].T, r2(p['in_b2']),
        r2(p['gn_in_w']), r2(p['gn_in_b']),
        p['cw1'], cb1row, p['cw2'].T, r2(p['cb2']),
        p['out_w1'].T, r2(p['out_b1']), p['out_w2'].T, r2(p['out_b2']),
        r2(p['gn_out_w']), r2(p['gn_out_b']),
        p['lin_w1'].T, r2(p['lin_b1']), p['lin_w2'].T, r2(p['lin_b2']),
        p['lt_w'].T, r2(p['lt_b']),
    ]
    in_specs = [_full(a.shape) for a in inputs]
    in_specs[1] = pl.BlockSpec((TB, 3), lambda i: (i, 0))
    in_specs[2] = pl.BlockSpec((TB, 9), lambda i: (i, 0))
    return pl.pallas_call(
        _layer_kernel,
        grid=(NB,),
        in_specs=in_specs,
        out_specs=_full((N, HID)),
        out_shape=jax.ShapeDtypeStruct((N, HID), jnp.float32),
        scratch_shapes=[
            pltpu.VMEM((N, HID), jnp.float32),
            pltpu.VMEM((N, HID), jnp.float32),
        ],
        compiler_params=pltpu.CompilerParams(
            dimension_semantics=("arbitrary",),
        ),
    )(*inputs)


@jax.jit
def kernel(features, points, nuv, params):
    scale = 1.0 / np.sqrt(2.0) / RADIUS
    pts_s = points * scale
    ptsT_s = pts_s.T
    nrmT = nuv[:, 0, :].T
    nuv9 = nuv.reshape(N, 9)
    sqrow = jnp.sum(ptsT_s * ptsT_s, axis=0, keepdims=True)
    x = features
    for p in params:
        x = _layer(x, pts_s, ptsT_s, sqrow, nrmT, nuv9, p)
    return x
